# Initial kernel scaffold; baseline (speedup 1.0000x reference)
import jax
import jax.numpy as jnp
from jax.experimental import pallas as pl


def kernel(ctrl_pts, Nu_uv, Nv_uv, uspan_uv, vspan_uv):
    raise NotImplementedError("write your pallas kernel here")


# speedup vs baseline: 18.2733x; 18.2733x over previous
import jax
import jax.numpy as jnp
from jax.experimental import pallas as pl
from jax.experimental.pallas import tpu as pltpu

_P = 3
_Q = 3


def _tc_body(uspan_ref, vspan_ref, nu_ref, nvt_ref, ctrl_ref, out_ref,
             bu_ref, bvt_ref):
    b = pl.program_id(0)

    @pl.when(b == 0)
    def _():
        col = jax.lax.broadcasted_iota(jnp.int32, (256, 64), 1)
        us = uspan_ref[...].reshape(256, 1)
        bu = jnp.zeros((256, 64), jnp.float32)
        for r in range(_P + 1):
            bu = bu + jnp.where(col == us - _P + r, nu_ref[:, r:r + 1], 0.0)
        bu_ref[...] = bu
        row = jax.lax.broadcasted_iota(jnp.int32, (64, 256), 0)
        vs = vspan_ref[...]
        bvt = jnp.zeros((64, 256), jnp.float32)
        for s in range(_Q + 1):
            bvt = bvt + jnp.where(row == vs - _Q + s, nvt_ref[s:s + 1, :], 0.0)
        bvt_ref[...] = bvt

    bu = bu_ref[...]
    bvt = bvt_ref[...]
    planes = []
    for d in range(4):
        a_d = jnp.dot(bu, ctrl_ref[0, d], preferred_element_type=jnp.float32)
        planes.append(jnp.dot(a_d, bvt, preferred_element_type=jnp.float32))
    rw = 1.0 / planes[3]
    for d in range(3):
        out_ref[0, d] = planes[d] * rw


def kernel(ctrl_pts, Nu_uv, Nv_uv, uspan_uv, vspan_uv):
    B, M, N, _ = ctrl_pts.shape
    G = Nu_uv.shape[0]
    ctrl_t = ctrl_pts.transpose(0, 3, 1, 2)
    uspan2 = uspan_uv.reshape(G, 1)
    vspan2 = vspan_uv.reshape(1, G)
    nvt = Nv_uv.T

    out = pl.pallas_call(
        _tc_body,
        grid=(B,),
        in_specs=[
            pl.BlockSpec((G, 1), lambda b: (0, 0)),
            pl.BlockSpec((1, G), lambda b: (0, 0)),
            pl.BlockSpec((G, 4), lambda b: (0, 0)),
            pl.BlockSpec((4, G), lambda b: (0, 0)),
            pl.BlockSpec((1, 4, M, N), lambda b: (b, 0, 0, 0)),
        ],
        out_specs=pl.BlockSpec((1, 3, G, G), lambda b: (b, 0, 0, 0)),
        out_shape=jax.ShapeDtypeStruct((B, 3, G, G), jnp.float32),
        scratch_shapes=[
            pltpu.VMEM((G, M), jnp.float32),
            pltpu.VMEM((N, G), jnp.float32),
        ],
    )(uspan2, vspan2, Nu_uv, nvt, ctrl_t)
    return out.transpose(0, 2, 3, 1)
